# baseline (speedup 1.0000x reference)
import functools

import jax
import jax.numpy as jnp
from jax import lax
from jax.experimental import pallas as pl
from jax.experimental.pallas import tpu as pltpu
from jax.experimental.pallas import tpu_sc as plsc

N = 10000
E = 320000
D = 128
H = 128
G = 64

NC = 2
NS = 16
NW = NC * NS
EPW = E // NW
CH = 125
NCHUNK = EPW // CH
IW = 8
NWIN = NCHUNK // IW
NP = 10240
RPT = NP // NS

_mesh_cache = []


def _MESH():
    if not _mesh_cache:
        _mesh_cache.append(plsc.VectorSubcoreMesh(
            core_axis_name="c", subcore_axis_name="s",
            num_cores=NC, num_subcores=NS))
    return _mesh_cache[0]


def _sc_degree_body(dstw, zeros128, ones128, out, idx_v, ones_v, acc, dsems):
    cid = lax.axis_index("c")
    sid = lax.axis_index("s")
    wid = cid * NS + sid
    pltpu.sync_copy(zeros128, acc.at[pl.ds(sid * RPT, RPT)])
    pltpu.sync_copy(ones128, ones_v)
    pltpu.sync_copy(dstw.at[wid], idx_v)
    plsc.subcore_barrier()

    def scat(g, b):
        pltpu.async_copy(ones_v, acc.at[idx_v.at[g]], dsems.at[b], add=True)

    def scat_wait(g, b):
        pltpu.make_async_copy(ones_v, acc.at[idx_v.at[g]], dsems.at[b]).wait()

    for b in range(4):
        scat(b, b)

    def quad(jj, carry):
        j = jj * 4
        for b in range(4):
            scat_wait(j - 4 + b, b)
            pltpu.async_copy(ones_v, acc.at[idx_v.at[j + b]], dsems.at[b],
                             add=True)
        return carry

    lax.fori_loop(1, NCHUNK // 4, quad, 0)
    for g in range(NCHUNK - 4, NCHUNK):
        scat_wait(g, g % 4)
    plsc.subcore_barrier()
    pltpu.sync_copy(acc.at[pl.ds(sid * RPT, RPT)],
                    out.at[cid, pl.ds(sid * RPT, RPT)])


def _sc_degree(dstw, zeros128, ones128):
    return pl.kernel(
        _sc_degree_body,
        out_type=jax.ShapeDtypeStruct((NC, NP, H), jnp.float32),
        mesh=_MESH(),
        scratch_types=[
            pltpu.VMEM((NCHUNK, CH), jnp.int32),
            pltpu.VMEM((CH, H), jnp.float32),
            pltpu.VMEM_SHARED((NP, H), jnp.float32),
            pltpu.SemaphoreType.DMA((4,)),
        ],
    )(dstw, zeros128, ones128)


NBUF = 2


def _sc_aggregate_body(hs, srcw, dstw, zeros128, out,
                       sidx_v, didx_v, rows0, rows1, acc, gsems, ssems, isem):
    cid = lax.axis_index("c")
    sid = lax.axis_index("s")
    wid = cid * NS + sid
    rows = (rows0, rows1)
    pltpu.sync_copy(zeros128, acc.at[pl.ds(sid * RPT, RPT)])
    pltpu.sync_copy(srcw.at[wid, pl.ds(0, IW)], sidx_v.at[0])
    pltpu.sync_copy(dstw.at[wid, pl.ds(0, IW)], didx_v.at[0])
    pltpu.async_copy(srcw.at[wid, pl.ds(IW, IW)], sidx_v.at[1], isem)
    pltpu.async_copy(dstw.at[wid, pl.ds(IW, IW)], didx_v.at[1], isem)
    plsc.subcore_barrier()

    def gath(p, l, b):
        pltpu.async_copy(hs.at[sidx_v.at[p, l]], rows[b], gsems.at[b])

    def gath_wait(b):
        pltpu.make_async_copy(hs.at[sidx_v.at[0, 0]], rows[b],
                              gsems.at[b]).wait()

    def scat(p, l, b):
        pltpu.async_copy(rows[b], acc.at[didx_v.at[p, l]], ssems.at[b],
                         add=True)

    def scat_wait(b):
        pltpu.make_async_copy(rows[b], acc.at[didx_v.at[0, 0]],
                              ssems.at[b]).wait()

    def idx_load(w, p):
        pltpu.async_copy(srcw.at[wid, pl.ds(w * IW, IW)], sidx_v.at[p], isem)
        pltpu.async_copy(dstw.at[wid, pl.ds(w * IW, IW)], didx_v.at[p], isem)

    def idx_wait(p):
        pltpu.make_async_copy(srcw.at[wid, pl.ds(0, IW)], sidx_v.at[p],
                              isem).wait()
        pltpu.make_async_copy(dstw.at[wid, pl.ds(0, IW)], didx_v.at[p],
                              isem).wait()

    gath(0, 0, 0)
    for w in range(NWIN):
        p = w % 2
        if w >= 1:
            scat_wait(1)
            if w <= NWIN - 2:
                idx_load(w + 1, 1 - p)
        gath(p, 1, 1)
        gath_wait(0)
        scat(p, 0, 0)

        def pair(ii, carry, p=p):
            l0 = ii * 2 + 1
            scat_wait(0)
            gath(p, l0 + 1, 0)
            gath_wait(1)
            scat(p, l0, 1)
            scat_wait(1)
            gath(p, l0 + 2, 1)
            gath_wait(0)
            scat(p, l0 + 1, 0)
            return carry

        lax.fori_loop(0, (IW - 2) // 2, pair, 0)

        scat_wait(0)
        if w < NWIN - 1:
            idx_wait(1 - p)
            gath(1 - p, 0, 0)
        gath_wait(1)
        scat(p, IW - 1, 1)

    scat_wait(1)
    plsc.subcore_barrier()
    pltpu.sync_copy(acc.at[pl.ds(sid * RPT, RPT)],
                    out.at[cid, pl.ds(sid * RPT, RPT)])


def _sc_aggregate(hs, srcw, dstw, zeros128):
    return pl.kernel(
        _sc_aggregate_body,
        out_type=jax.ShapeDtypeStruct((NC, NP, H), jnp.float32),
        mesh=_MESH(),
        scratch_types=[
            pltpu.VMEM((2, IW, CH), jnp.int32),
            pltpu.VMEM((2, IW, CH), jnp.int32),
            pltpu.VMEM((CH, H), jnp.float32),
            pltpu.VMEM((CH, H), jnp.float32),
            pltpu.VMEM_SHARED((NP, H), jnp.float32),
            pltpu.SemaphoreType.DMA((NBUF,)),
            pltpu.SemaphoreType.DMA((NBUF,)),
            pltpu.SemaphoreType.DMA,
        ],
    )(hs, srcw, dstw, zeros128)


BLK = 1000
NBLK = N // BLK
_full = lambda shape: pl.BlockSpec(shape, lambda i: (0,) * len(shape))
_rows = lambda w: pl.BlockSpec((BLK, w), lambda i: (i, 0))


def _tc_pre_body(x_ref, w1_ref, wp_ref, bp_ref, degp_ref,
                 hs1_ref, xp_ref, dinvc_ref):
    deg = degp_ref[0, :, 0:1] + degp_ref[1, :, 0:1] + 1.0
    dinv = lax.rsqrt(deg)
    x = x_ref[...]
    h = lax.dot_general(x, w1_ref[...], (((1,), (1,)), ((), ())),
                        preferred_element_type=jnp.float32)
    hs1_ref[...] = dinv * h
    xp_ref[...] = lax.dot_general(x, wp_ref[...], (((1,), (1,)), ((), ())),
                                  preferred_element_type=jnp.float32) + bp_ref[...]
    dinvc_ref[...] = jnp.broadcast_to(dinv, (BLK, 16))


def _tc_pre(x, W1, Wp, bp, degp):
    return pl.pallas_call(
        _tc_pre_body,
        grid=(NBLK,),
        in_specs=[
            _rows(D),
            _full((H, D)),
            _full((H, D)),
            _full((1, H)),
            pl.BlockSpec((NC, BLK, H), lambda i: (0, i, 0)),
        ],
        out_specs=[_rows(H), _rows(H), _rows(16)],
        out_shape=[
            jax.ShapeDtypeStruct((N, H), jnp.float32),
            jax.ShapeDtypeStruct((N, H), jnp.float32),
            jax.ShapeDtypeStruct((N, 16), jnp.float32),
        ],
    )(x, W1, Wp, bp, degp)


def _tc_mid_body(agg_ref, hs_ref, dinvc_ref, b_ref, wn_ref, xp_ref,
                 out_ref, *, with_xp):
    dinv = dinvc_ref[:, 0:1]
    t = agg_ref[0] + agg_ref[1] + hs_ref[...]
    t = jnp.maximum(dinv * t + b_ref[...], 0.0)
    if with_xp:
        t = t + xp_ref[...]
    else:
        t = t + t
    out_ref[...] = dinv * lax.dot_general(
        t, wn_ref[...], (((1,), (1,)), ((), ())),
        preferred_element_type=jnp.float32)


def _tc_mid(agg, hs, dinvc, b, Wn, xp, with_xp):
    return pl.pallas_call(
        functools.partial(_tc_mid_body, with_xp=with_xp),
        grid=(NBLK,),
        in_specs=[
            pl.BlockSpec((NC, BLK, H), lambda i: (0, i, 0)),
            _rows(H),
            _rows(16),
            _full((1, H)),
            _full((H, H)),
            _rows(H),
        ],
        out_specs=_rows(H),
        out_shape=jax.ShapeDtypeStruct((N, H), jnp.float32),
    )(agg, hs, dinvc, b, Wn, xp)


def _tc_poolhead_body(agg_ref, hs_ref, dinvc_ref, b_ref, batch_ref,
                      doc_ref, wd_ref, bd_ref, ga_ref, gb_ref, ba_ref,
                      bb_ref, wfa_ref, wfb_ref, bf_ref, wto_ref, bto_ref,
                      out_ref, sums_ref, cnt_ref):
    dinv = dinvc_ref[:, 0:1]
    t = agg_ref[0] + agg_ref[1] + hs_ref[...]
    h3 = jnp.maximum(dinv * t + b_ref[...], 0.0)
    h3 = h3 + h3
    bblk = batch_ref[0]
    giota = lax.broadcasted_iota(jnp.int32, (G, BLK), 0)
    oh = (bblk == giota).astype(jnp.float32)
    s = lax.dot_general(oh, h3, (((1,), (0,)), ((), ())),
                        preferred_element_type=jnp.float32)
    c = jnp.broadcast_to(jnp.sum(oh, axis=1, keepdims=True), (G, H))

    @pl.when(pl.program_id(0) == 0)
    def _():
        sums_ref[...] = jnp.zeros_like(sums_ref)
        cnt_ref[...] = jnp.zeros_like(cnt_ref)

    sums_ref[...] += s
    cnt_ref[...] += c

    @pl.when(pl.program_id(0) == NBLK - 1)
    def _():
        pooled = sums_ref[...] / jnp.maximum(cnt_ref[...], 1.0)
        de = jnp.maximum(
            lax.dot_general(doc_ref[...], wd_ref[...],
                            (((1,), (1,)), ((), ())),
                            preferred_element_type=jnp.float32)
            + bd_ref[...], 0.0)
        two_h = 2.0 * H
        mu = (jnp.sum(pooled, axis=1, keepdims=True)
              + jnp.sum(de, axis=1, keepdims=True)) / two_h
        pc = pooled - mu
        dc = de - mu
        var = (jnp.sum(pc * pc, axis=1, keepdims=True)
               + jnp.sum(dc * dc, axis=1, keepdims=True)) / two_h
        inv = lax.rsqrt(var + 1e-5)
        pn = pc * inv * ga_ref[...] + ba_ref[...]
        dn = dc * inv * gb_ref[...] + bb_ref[...]
        f = jnp.maximum(
            lax.dot_general(pn, wfa_ref[...], (((1,), (1,)), ((), ())),
                            preferred_element_type=jnp.float32)
            + lax.dot_general(dn, wfb_ref[...], (((1,), (1,)), ((), ())),
                              preferred_element_type=jnp.float32)
            + bf_ref[...], 0.0)
        out_ref[...] = lax.dot_general(
            f, wto_ref[...], (((1,), (1,)), ((), ())),
            preferred_element_type=jnp.float32) + bto_ref[...]


def _tc_poolhead(agg, hs, dinvc, b, batchr, doc, Wd, bd, ga, gb, ba, bb,
                 Wfa, Wfb, bf, Wto, bto):
    DOC = doc.shape[1]
    return pl.pallas_call(
        _tc_poolhead_body,
        grid=(NBLK,),
        in_specs=[
            pl.BlockSpec((NC, BLK, H), lambda i: (0, i, 0)),
            _rows(H),
            _rows(16),
            _full((1, H)),
            pl.BlockSpec((1, 1, BLK), lambda i: (i, 0, 0)),
            _full((G, DOC)), _full((H, DOC)), _full((1, H)),
            _full((1, H)), _full((1, H)), _full((1, H)), _full((1, H)),
            _full((H, H)), _full((H, H)), _full((1, H)),
            _full((16, H)), _full((1, 16)),
        ],
        out_specs=_full((G, 16)),
        out_shape=jax.ShapeDtypeStruct((G, 16), jnp.float32),
        scratch_shapes=[
            pltpu.VMEM((G, H), jnp.float32),
            pltpu.VMEM((G, H), jnp.float32),
        ],
    )(agg, hs, dinvc, b, batchr, doc, Wd, bd, ga, gb, ba, bb,
      Wfa, Wfb, bf, Wto, bto)


def kernel(x, edge_index, batch, doc_features, W1, b1, W2, b2, W3, b3,
           Wp, bp, Wd, bd, gamma, beta, Wf, bf, Wt, bt, Wtm, btm):
    f32 = jnp.float32
    srcw = edge_index[0].astype(jnp.int32).reshape(NW, NCHUNK, CH)
    dstw = edge_index[1].astype(jnp.int32).reshape(NW, NCHUNK, CH)
    batchr = batch.astype(jnp.int32).reshape(NBLK, 1, BLK)
    zeros128 = jnp.zeros((RPT, H), f32)
    ones128 = jnp.ones((CH, H), f32)

    b1r = b1.reshape(1, H)
    b2r = b2.reshape(1, H)
    b3r = b3.reshape(1, H)
    bpr = bp.reshape(1, H)
    bdr = bd.reshape(1, H)
    bfr = bf.reshape(1, H)
    ga, gb = gamma[:H].reshape(1, H), gamma[H:].reshape(1, H)
    ba, bb = beta[:H].reshape(1, H), beta[H:].reshape(1, H)
    Wfa, Wfb = Wf[:, :H], Wf[:, H:]
    Wto = jnp.concatenate(
        [Wt, Wtm, jnp.zeros((16 - Wt.shape[0] - 1, H), f32)], axis=0)
    bto = jnp.concatenate(
        [bt, btm, jnp.zeros((16 - bt.shape[0] - 1,), f32)]).reshape(1, 16)

    degp = _sc_degree(dstw, zeros128, ones128)
    hs1, xp, dinvc = _tc_pre(x, W1, Wp, bpr, degp)
    agg1 = _sc_aggregate(hs1, srcw, dstw, zeros128)
    hs2 = _tc_mid(agg1, hs1, dinvc, b1r, W2, xp, with_xp=True)
    agg2 = _sc_aggregate(hs2, srcw, dstw, zeros128)
    hs3 = _tc_mid(agg2, hs2, dinvc, b2r, W3, xp, with_xp=False)
    agg3 = _sc_aggregate(hs3, srcw, dstw, zeros128)
    out = _tc_poolhead(agg3, hs3, dinvc, b3r, batchr, doc_features, Wd,
                       bdr, ga, gb, ba, bb, Wfa, Wfb, bfr, Wto, bto)
    task = out[:, :10]
    time = out[:, 10:11]
    return (task, time)


# speedup vs baseline: 1.0326x; 1.0326x over previous
import functools

import jax
import jax.numpy as jnp
from jax import lax
from jax.experimental import pallas as pl
from jax.experimental.pallas import tpu as pltpu
from jax.experimental.pallas import tpu_sc as plsc

N = 10000
E = 320000
D = 128
H = 128
G = 64

NC = 2
NS = 16
NW = NC * NS
EPW = E // NW
CH = 125
NCHUNK = EPW // CH
IW = 8
NWIN = NCHUNK // IW
NP = 10240
RPT = NP // NS

_mesh_cache = []


def _MESH():
    if not _mesh_cache:
        _mesh_cache.append(plsc.VectorSubcoreMesh(
            core_axis_name="c", subcore_axis_name="s",
            num_cores=NC, num_subcores=NS))
    return _mesh_cache[0]


def _sc_degree_body(ei4, zeros128, ones128, out, idx_v, ones_v, acc, dsems):
    cid = lax.axis_index("c")
    sid = lax.axis_index("s")
    wid = cid * NS + sid
    pltpu.sync_copy(zeros128, acc.at[pl.ds(sid * RPT, RPT)])
    pltpu.sync_copy(ones128, ones_v)
    pltpu.sync_copy(ei4.at[1, wid], idx_v)
    plsc.subcore_barrier()

    def scat(g, b):
        pltpu.async_copy(ones_v, acc.at[idx_v.at[g]], dsems.at[b], add=True)

    def scat_wait(g, b):
        pltpu.make_async_copy(ones_v, acc.at[idx_v.at[g]], dsems.at[b]).wait()

    for b in range(4):
        scat(b, b)

    def quad(jj, carry):
        j = jj * 4
        for b in range(4):
            scat_wait(j - 4 + b, b)
            pltpu.async_copy(ones_v, acc.at[idx_v.at[j + b]], dsems.at[b],
                             add=True)
        return carry

    lax.fori_loop(1, NCHUNK // 4, quad, 0)
    for g in range(NCHUNK - 4, NCHUNK):
        scat_wait(g, g % 4)
    plsc.subcore_barrier()
    pltpu.sync_copy(acc.at[pl.ds(sid * RPT, RPT)],
                    out.at[cid, pl.ds(sid * RPT, RPT)])


def _sc_degree(ei4, zeros128, ones128):
    return pl.kernel(
        _sc_degree_body,
        out_type=jax.ShapeDtypeStruct((NC, NP, H), jnp.float32),
        mesh=_MESH(),
        scratch_types=[
            pltpu.VMEM((NCHUNK, CH), jnp.int32),
            pltpu.VMEM((CH, H), jnp.float32),
            pltpu.VMEM_SHARED((NP, H), jnp.float32),
            pltpu.SemaphoreType.DMA((4,)),
        ],
    )(ei4, zeros128, ones128)


NBUF = 2


def _sc_aggregate_body(hs, ei4, zeros128, out,
                       sidx_v, didx_v, rows0, rows1, acc, gsems, ssems, isem):
    cid = lax.axis_index("c")
    sid = lax.axis_index("s")
    wid = cid * NS + sid
    rows = (rows0, rows1)
    pltpu.sync_copy(zeros128, acc.at[pl.ds(sid * RPT, RPT)])
    pltpu.sync_copy(ei4.at[0, wid, pl.ds(0, IW)], sidx_v.at[0])
    pltpu.sync_copy(ei4.at[1, wid, pl.ds(0, IW)], didx_v.at[0])
    pltpu.async_copy(ei4.at[0, wid, pl.ds(IW, IW)], sidx_v.at[1], isem)
    pltpu.async_copy(ei4.at[1, wid, pl.ds(IW, IW)], didx_v.at[1], isem)
    plsc.subcore_barrier()

    def gath(p, l, b):
        pltpu.async_copy(hs.at[sidx_v.at[p, l]], rows[b], gsems.at[b])

    def gath_wait(b):
        pltpu.make_async_copy(hs.at[sidx_v.at[0, 0]], rows[b],
                              gsems.at[b]).wait()

    def scat(p, l, b):
        pltpu.async_copy(rows[b], acc.at[didx_v.at[p, l]], ssems.at[b],
                         add=True)

    def scat_wait(b):
        pltpu.make_async_copy(rows[b], acc.at[didx_v.at[0, 0]],
                              ssems.at[b]).wait()

    def idx_load(w, p):
        pltpu.async_copy(ei4.at[0, wid, pl.ds(w * IW, IW)], sidx_v.at[p], isem)
        pltpu.async_copy(ei4.at[1, wid, pl.ds(w * IW, IW)], didx_v.at[p], isem)

    def idx_wait(p):
        pltpu.make_async_copy(ei4.at[0, wid, pl.ds(0, IW)], sidx_v.at[p],
                              isem).wait()
        pltpu.make_async_copy(ei4.at[1, wid, pl.ds(0, IW)], didx_v.at[p],
                              isem).wait()

    gath(0, 0, 0)
    for w in range(NWIN):
        p = w % 2
        if w >= 1:
            scat_wait(1)
            if w <= NWIN - 2:
                idx_load(w + 1, 1 - p)
        gath(p, 1, 1)
        gath_wait(0)
        scat(p, 0, 0)

        def pair(ii, carry, p=p):
            l0 = ii * 2 + 1
            scat_wait(0)
            gath(p, l0 + 1, 0)
            gath_wait(1)
            scat(p, l0, 1)
            scat_wait(1)
            gath(p, l0 + 2, 1)
            gath_wait(0)
            scat(p, l0 + 1, 0)
            return carry

        lax.fori_loop(0, (IW - 2) // 2, pair, 0)

        scat_wait(0)
        if w < NWIN - 1:
            idx_wait(1 - p)
            gath(1 - p, 0, 0)
        gath_wait(1)
        scat(p, IW - 1, 1)

    scat_wait(1)
    plsc.subcore_barrier()
    pltpu.sync_copy(acc.at[pl.ds(sid * RPT, RPT)],
                    out.at[cid, pl.ds(sid * RPT, RPT)])


def _sc_aggregate(hs, ei4, zeros128):
    return pl.kernel(
        _sc_aggregate_body,
        out_type=jax.ShapeDtypeStruct((NC, NP, H), jnp.float32),
        mesh=_MESH(),
        scratch_types=[
            pltpu.VMEM((2, IW, CH), jnp.int32),
            pltpu.VMEM((2, IW, CH), jnp.int32),
            pltpu.VMEM((CH, H), jnp.float32),
            pltpu.VMEM((CH, H), jnp.float32),
            pltpu.VMEM_SHARED((NP, H), jnp.float32),
            pltpu.SemaphoreType.DMA((NBUF,)),
            pltpu.SemaphoreType.DMA((NBUF,)),
            pltpu.SemaphoreType.DMA,
        ],
    )(hs, ei4, zeros128)


BLK = 2000
NBLK = N // BLK
_full = lambda shape: pl.BlockSpec(shape, lambda i: (0,) * len(shape))
_rows = lambda w: pl.BlockSpec((BLK, w), lambda i: (i, 0))


def _tc_pre_body(x_ref, w1_ref, wp_ref, bp_ref, degp_ref,
                 hs1_ref, xp_ref, dinvc_ref):
    deg = degp_ref[0, :, 0:1] + degp_ref[1, :, 0:1] + 1.0
    dinv = lax.rsqrt(deg)
    x = x_ref[...]
    h = lax.dot_general(x, w1_ref[...], (((1,), (1,)), ((), ())),
                        preferred_element_type=jnp.float32)
    hs1_ref[...] = dinv * h
    xp_ref[...] = lax.dot_general(x, wp_ref[...], (((1,), (1,)), ((), ())),
                                  preferred_element_type=jnp.float32) + bp_ref[...]
    dinvc_ref[...] = jnp.broadcast_to(dinv, (BLK, 16))


def _tc_pre(x, W1, Wp, bp, degp):
    return pl.pallas_call(
        _tc_pre_body,
        grid=(NBLK,),
        in_specs=[
            _rows(D),
            _full((H, D)),
            _full((H, D)),
            _full((1, H)),
            pl.BlockSpec((NC, BLK, H), lambda i: (0, i, 0)),
        ],
        out_specs=[_rows(H), _rows(H), _rows(16)],
        out_shape=[
            jax.ShapeDtypeStruct((N, H), jnp.float32),
            jax.ShapeDtypeStruct((N, H), jnp.float32),
            jax.ShapeDtypeStruct((N, 16), jnp.float32),
        ],
    )(x, W1, Wp, bp, degp)


def _tc_mid_body(agg_ref, hs_ref, dinvc_ref, b_ref, wn_ref, xp_ref,
                 out_ref, *, with_xp):
    dinv = dinvc_ref[:, 0:1]
    t = agg_ref[0] + agg_ref[1] + hs_ref[...]
    t = jnp.maximum(dinv * t + b_ref[...], 0.0)
    if with_xp:
        t = t + xp_ref[...]
    else:
        t = t + t
    out_ref[...] = dinv * lax.dot_general(
        t, wn_ref[...], (((1,), (1,)), ((), ())),
        preferred_element_type=jnp.float32)


def _tc_mid(agg, hs, dinvc, b, Wn, xp, with_xp):
    return pl.pallas_call(
        functools.partial(_tc_mid_body, with_xp=with_xp),
        grid=(NBLK,),
        in_specs=[
            pl.BlockSpec((NC, BLK, H), lambda i: (0, i, 0)),
            _rows(H),
            _rows(16),
            _full((1, H)),
            _full((H, H)),
            _rows(H),
        ],
        out_specs=_rows(H),
        out_shape=jax.ShapeDtypeStruct((N, H), jnp.float32),
    )(agg, hs, dinvc, b, Wn, xp)


def _tc_poolhead_body(agg_ref, hs_ref, dinvc_ref, b_ref, batch_ref,
                      doc_ref, wd_ref, bd_ref, ga_ref, gb_ref, ba_ref,
                      bb_ref, wfa_ref, wfb_ref, bf_ref, wto_ref, bto_ref,
                      out_ref, sums_ref, cnt_ref):
    dinv = dinvc_ref[:, 0:1]
    t = agg_ref[0] + agg_ref[1] + hs_ref[...]
    h3 = jnp.maximum(dinv * t + b_ref[...], 0.0)
    h3 = h3 + h3
    bblk = batch_ref[0]
    giota = lax.broadcasted_iota(jnp.int32, (G, BLK), 0)
    oh = (bblk == giota).astype(jnp.float32)
    s = lax.dot_general(oh, h3, (((1,), (0,)), ((), ())),
                        preferred_element_type=jnp.float32)
    c = jnp.broadcast_to(jnp.sum(oh, axis=1, keepdims=True), (G, H))

    @pl.when(pl.program_id(0) == 0)
    def _():
        sums_ref[...] = jnp.zeros_like(sums_ref)
        cnt_ref[...] = jnp.zeros_like(cnt_ref)

    sums_ref[...] += s
    cnt_ref[...] += c

    @pl.when(pl.program_id(0) == NBLK - 1)
    def _():
        pooled = sums_ref[...] / jnp.maximum(cnt_ref[...], 1.0)
        de = jnp.maximum(
            lax.dot_general(doc_ref[...], wd_ref[...],
                            (((1,), (1,)), ((), ())),
                            preferred_element_type=jnp.float32)
            + bd_ref[...], 0.0)
        two_h = 2.0 * H
        mu = (jnp.sum(pooled, axis=1, keepdims=True)
              + jnp.sum(de, axis=1, keepdims=True)) / two_h
        pc = pooled - mu
        dc = de - mu
        var = (jnp.sum(pc * pc, axis=1, keepdims=True)
               + jnp.sum(dc * dc, axis=1, keepdims=True)) / two_h
        inv = lax.rsqrt(var + 1e-5)
        pn = pc * inv * ga_ref[...] + ba_ref[...]
        dn = dc * inv * gb_ref[...] + bb_ref[...]
        f = jnp.maximum(
            lax.dot_general(pn, wfa_ref[...], (((1,), (1,)), ((), ())),
                            preferred_element_type=jnp.float32)
            + lax.dot_general(dn, wfb_ref[...], (((1,), (1,)), ((), ())),
                              preferred_element_type=jnp.float32)
            + bf_ref[...], 0.0)
        out_ref[...] = lax.dot_general(
            f, wto_ref[...], (((1,), (1,)), ((), ())),
            preferred_element_type=jnp.float32) + bto_ref[...]


def _tc_poolhead(agg, hs, dinvc, b, batchr, doc, Wd, bd, ga, gb, ba, bb,
                 Wfa, Wfb, bf, Wto, bto):
    DOC = doc.shape[1]
    return pl.pallas_call(
        _tc_poolhead_body,
        grid=(NBLK,),
        in_specs=[
            pl.BlockSpec((NC, BLK, H), lambda i: (0, i, 0)),
            _rows(H),
            _rows(16),
            _full((1, H)),
            pl.BlockSpec((1, 1, BLK), lambda i: (i, 0, 0)),
            _full((G, DOC)), _full((H, DOC)), _full((1, H)),
            _full((1, H)), _full((1, H)), _full((1, H)), _full((1, H)),
            _full((H, H)), _full((H, H)), _full((1, H)),
            _full((16, H)), _full((1, 16)),
        ],
        out_specs=_full((G, 16)),
        out_shape=jax.ShapeDtypeStruct((G, 16), jnp.float32),
        scratch_shapes=[
            pltpu.VMEM((G, H), jnp.float32),
            pltpu.VMEM((G, H), jnp.float32),
        ],
    )(agg, hs, dinvc, b, batchr, doc, Wd, bd, ga, gb, ba, bb,
      Wfa, Wfb, bf, Wto, bto)


def kernel(x, edge_index, batch, doc_features, W1, b1, W2, b2, W3, b3,
           Wp, bp, Wd, bd, gamma, beta, Wf, bf, Wt, bt, Wtm, btm):
    f32 = jnp.float32
    ei4 = edge_index.astype(jnp.int32).reshape(2, NW, NCHUNK, CH)
    batchr = batch.astype(jnp.int32).reshape(NBLK, 1, BLK)
    zeros128 = jnp.zeros((RPT, H), f32)
    ones128 = jnp.ones((CH, H), f32)

    b1r = b1.reshape(1, H)
    b2r = b2.reshape(1, H)
    b3r = b3.reshape(1, H)
    bpr = bp.reshape(1, H)
    bdr = bd.reshape(1, H)
    bfr = bf.reshape(1, H)
    ga, gb = gamma[:H].reshape(1, H), gamma[H:].reshape(1, H)
    ba, bb = beta[:H].reshape(1, H), beta[H:].reshape(1, H)
    Wfa, Wfb = Wf[:, :H], Wf[:, H:]
    Wto = jnp.concatenate(
        [Wt, Wtm, jnp.zeros((16 - Wt.shape[0] - 1, H), f32)], axis=0)
    bto = jnp.concatenate(
        [bt, btm, jnp.zeros((16 - bt.shape[0] - 1,), f32)]).reshape(1, 16)

    degp = _sc_degree(ei4, zeros128, ones128)
    hs1, xp, dinvc = _tc_pre(x, W1, Wp, bpr, degp)
    agg1 = _sc_aggregate(hs1, ei4, zeros128)
    hs2 = _tc_mid(agg1, hs1, dinvc, b1r, W2, xp, with_xp=True)
    agg2 = _sc_aggregate(hs2, ei4, zeros128)
    hs3 = _tc_mid(agg2, hs2, dinvc, b2r, W3, xp, with_xp=False)
    agg3 = _sc_aggregate(hs3, ei4, zeros128)
    out = _tc_poolhead(agg3, hs3, dinvc, b3r, batchr, doc_features, Wd,
                       bdr, ga, gb, ba, bb, Wfa, Wfb, bfr, Wto, bto)
    task = out[:, :10]
    time = out[:, 10:11]
    return (task, time)


# speedup vs baseline: 1.1933x; 1.1557x over previous
import functools

import jax
import jax.numpy as jnp
from jax import lax
from jax.experimental import pallas as pl
from jax.experimental.pallas import tpu as pltpu
from jax.experimental.pallas import tpu_sc as plsc

N = 10000
E = 320000
D = 128
H = 128
G = 64

NC = 2
NS = 16
NW = NC * NS
EPW = E // NW
CH = 125
NCHUNK = EPW // CH
IW = 8
NWIN = NCHUNK // IW
NP = 10240
RPT = NP // NS

_mesh_cache = []


def _MESH():
    if not _mesh_cache:
        _mesh_cache.append(plsc.VectorSubcoreMesh(
            core_axis_name="c", subcore_axis_name="s",
            num_cores=NC, num_subcores=NS))
    return _mesh_cache[0]


def _sc_degree_body(ei2, out, idx1, cnt_v, seg_v, seg2_v, rep_v):
    cid = lax.axis_index("c")
    sid = lax.axis_index("s")
    wid = cid * NS + sid
    for i in range(NP // 16):
        cnt_v[pl.ds(i * 16, 16)] = jnp.zeros((16,), jnp.float32)
    pltpu.sync_copy(ei2.at[1, wid], idx1)
    ones = jnp.ones((16,), jnp.float32)
    for i in range(EPW // 16):
        iv = idx1[pl.ds(i * 16, 16)]
        plsc.addupdate_scatter(cnt_v, [iv], ones)
    pltpu.sync_copy(cnt_v, seg_v.at[sid])
    plsc.subcore_barrier()
    pltpu.sync_copy(seg_v.at[:, pl.ds(sid * RPT, RPT)], seg2_v)
    col0 = jnp.zeros((16,), jnp.int32)
    for g in range(RPT // 64):
        for j in range(4):
            base = g * 64 + j * 16
            t = seg2_v[0, pl.ds(base, 16)]
            for tt in range(1, NS):
                t = t + seg2_v[tt, pl.ds(base, 16)]
            ridx = lax.iota(jnp.int32, 16) + j * 16
            plsc.store_scatter(rep_v, [ridx, col0], t)
        pltpu.sync_copy(
            rep_v, out.at[cid, pl.ds(sid * RPT + g * 64, 64)])


def _sc_degree(ei2):
    return pl.kernel(
        _sc_degree_body,
        out_type=jax.ShapeDtypeStruct((NC, NP, H), jnp.float32),
        mesh=_MESH(),
        compiler_params=pltpu.CompilerParams(needs_layout_passes=False),
        scratch_types=[
            pltpu.VMEM((EPW,), jnp.int32),
            pltpu.VMEM((NP,), jnp.float32),
            pltpu.VMEM_SHARED((NS, NP), jnp.float32),
            pltpu.VMEM((NS, RPT), jnp.float32),
            pltpu.VMEM((64, H), jnp.float32),
        ],
    )(ei2)


NBUF = 2


def _sc_aggregate_body(hs, ei4, zeros128, out,
                       sidx_v, didx_v, rows0, rows1, acc, gsems, ssems, isem):
    cid = lax.axis_index("c")
    sid = lax.axis_index("s")
    wid = cid * NS + sid
    rows = (rows0, rows1)
    pltpu.sync_copy(zeros128, acc.at[pl.ds(sid * RPT, RPT)])
    pltpu.sync_copy(ei4.at[0, wid, pl.ds(0, IW)], sidx_v.at[0])
    pltpu.sync_copy(ei4.at[1, wid, pl.ds(0, IW)], didx_v.at[0])
    pltpu.async_copy(ei4.at[0, wid, pl.ds(IW, IW)], sidx_v.at[1], isem)
    pltpu.async_copy(ei4.at[1, wid, pl.ds(IW, IW)], didx_v.at[1], isem)
    plsc.subcore_barrier()

    def gath(p, l, b):
        pltpu.async_copy(hs.at[sidx_v.at[p, l]], rows[b], gsems.at[b])

    def gath_wait(b):
        pltpu.make_async_copy(hs.at[sidx_v.at[0, 0]], rows[b],
                              gsems.at[b]).wait()

    def scat(p, l, b):
        pltpu.async_copy(rows[b], acc.at[didx_v.at[p, l]], ssems.at[b],
                         add=True)

    def scat_wait(b):
        pltpu.make_async_copy(rows[b], acc.at[didx_v.at[0, 0]],
                              ssems.at[b]).wait()

    def idx_load(w, p):
        pltpu.async_copy(ei4.at[0, wid, pl.ds(w * IW, IW)], sidx_v.at[p], isem)
        pltpu.async_copy(ei4.at[1, wid, pl.ds(w * IW, IW)], didx_v.at[p], isem)

    def idx_wait(p):
        pltpu.make_async_copy(ei4.at[0, wid, pl.ds(0, IW)], sidx_v.at[p],
                              isem).wait()
        pltpu.make_async_copy(ei4.at[1, wid, pl.ds(0, IW)], didx_v.at[p],
                              isem).wait()

    gath(0, 0, 0)
    for w in range(NWIN):
        p = w % 2
        if w >= 1:
            scat_wait(1)
            if w <= NWIN - 2:
                idx_load(w + 1, 1 - p)
        gath(p, 1, 1)
        gath_wait(0)
        scat(p, 0, 0)

        def pair(ii, carry, p=p):
            l0 = ii * 2 + 1
            scat_wait(0)
            gath(p, l0 + 1, 0)
            gath_wait(1)
            scat(p, l0, 1)
            scat_wait(1)
            gath(p, l0 + 2, 1)
            gath_wait(0)
            scat(p, l0 + 1, 0)
            return carry

        lax.fori_loop(0, (IW - 2) // 2, pair, 0)

        scat_wait(0)
        if w < NWIN - 1:
            idx_wait(1 - p)
            gath(1 - p, 0, 0)
        gath_wait(1)
        scat(p, IW - 1, 1)

    scat_wait(1)
    plsc.subcore_barrier()
    pltpu.sync_copy(acc.at[pl.ds(sid * RPT, RPT)],
                    out.at[cid, pl.ds(sid * RPT, RPT)])


def _sc_aggregate(hs, ei4, zeros128):
    return pl.kernel(
        _sc_aggregate_body,
        out_type=jax.ShapeDtypeStruct((NC, NP, H), jnp.float32),
        mesh=_MESH(),
        scratch_types=[
            pltpu.VMEM((2, IW, CH), jnp.int32),
            pltpu.VMEM((2, IW, CH), jnp.int32),
            pltpu.VMEM((CH, H), jnp.float32),
            pltpu.VMEM((CH, H), jnp.float32),
            pltpu.VMEM_SHARED((NP, H), jnp.float32),
            pltpu.SemaphoreType.DMA((NBUF,)),
            pltpu.SemaphoreType.DMA((NBUF,)),
            pltpu.SemaphoreType.DMA,
        ],
    )(hs, ei4, zeros128)


BLK = 2000
NBLK = N // BLK
_full = lambda shape: pl.BlockSpec(shape, lambda i: (0,) * len(shape))
_rows = lambda w: pl.BlockSpec((BLK, w), lambda i: (i, 0))


def _tc_pre_body(x_ref, w1_ref, wp_ref, bp_ref, degp_ref,
                 hs1_ref, xp_ref, dinvc_ref):
    deg = degp_ref[0, :, 0:1] + degp_ref[1, :, 0:1] + 1.0
    dinv = lax.rsqrt(deg)
    x = x_ref[...]
    h = lax.dot_general(x, w1_ref[...], (((1,), (1,)), ((), ())),
                        preferred_element_type=jnp.float32)
    hs1_ref[...] = dinv * h
    xp_ref[...] = lax.dot_general(x, wp_ref[...], (((1,), (1,)), ((), ())),
                                  preferred_element_type=jnp.float32) + bp_ref[...]
    dinvc_ref[...] = jnp.broadcast_to(dinv, (BLK, 16))


def _tc_pre(x, W1, Wp, bp, degp):
    return pl.pallas_call(
        _tc_pre_body,
        grid=(NBLK,),
        in_specs=[
            _rows(D),
            _full((H, D)),
            _full((H, D)),
            _full((1, H)),
            pl.BlockSpec((NC, BLK, H), lambda i: (0, i, 0)),
        ],
        out_specs=[_rows(H), _rows(H), _rows(16)],
        out_shape=[
            jax.ShapeDtypeStruct((N, H), jnp.float32),
            jax.ShapeDtypeStruct((N, H), jnp.float32),
            jax.ShapeDtypeStruct((N, 16), jnp.float32),
        ],
    )(x, W1, Wp, bp, degp)


def _tc_mid_body(agg_ref, hs_ref, dinvc_ref, b_ref, wn_ref, xp_ref,
                 out_ref, *, with_xp):
    dinv = dinvc_ref[:, 0:1]
    t = agg_ref[0] + agg_ref[1] + hs_ref[...]
    t = jnp.maximum(dinv * t + b_ref[...], 0.0)
    if with_xp:
        t = t + xp_ref[...]
    else:
        t = t + t
    out_ref[...] = dinv * lax.dot_general(
        t, wn_ref[...], (((1,), (1,)), ((), ())),
        preferred_element_type=jnp.float32)


def _tc_mid(agg, hs, dinvc, b, Wn, xp, with_xp):
    return pl.pallas_call(
        functools.partial(_tc_mid_body, with_xp=with_xp),
        grid=(NBLK,),
        in_specs=[
            pl.BlockSpec((NC, BLK, H), lambda i: (0, i, 0)),
            _rows(H),
            _rows(16),
            _full((1, H)),
            _full((H, H)),
            _rows(H),
        ],
        out_specs=_rows(H),
        out_shape=jax.ShapeDtypeStruct((N, H), jnp.float32),
    )(agg, hs, dinvc, b, Wn, xp)


def _tc_poolhead_body(agg_ref, hs_ref, dinvc_ref, b_ref, batch_ref,
                      doc_ref, wd_ref, bd_ref, ga_ref, gb_ref, ba_ref,
                      bb_ref, wfa_ref, wfb_ref, bf_ref, wto_ref, bto_ref,
                      out_ref, sums_ref, cnt_ref):
    dinv = dinvc_ref[:, 0:1]
    t = agg_ref[0] + agg_ref[1] + hs_ref[...]
    h3 = jnp.maximum(dinv * t + b_ref[...], 0.0)
    h3 = h3 + h3
    bblk = batch_ref[0]
    giota = lax.broadcasted_iota(jnp.int32, (G, BLK), 0)
    oh = (bblk == giota).astype(jnp.float32)
    s = lax.dot_general(oh, h3, (((1,), (0,)), ((), ())),
                        preferred_element_type=jnp.float32)
    c = jnp.broadcast_to(jnp.sum(oh, axis=1, keepdims=True), (G, H))

    @pl.when(pl.program_id(0) == 0)
    def _():
        sums_ref[...] = jnp.zeros_like(sums_ref)
        cnt_ref[...] = jnp.zeros_like(cnt_ref)

    sums_ref[...] += s
    cnt_ref[...] += c

    @pl.when(pl.program_id(0) == NBLK - 1)
    def _():
        pooled = sums_ref[...] / jnp.maximum(cnt_ref[...], 1.0)
        de = jnp.maximum(
            lax.dot_general(doc_ref[...], wd_ref[...],
                            (((1,), (1,)), ((), ())),
                            preferred_element_type=jnp.float32)
            + bd_ref[...], 0.0)
        two_h = 2.0 * H
        mu = (jnp.sum(pooled, axis=1, keepdims=True)
              + jnp.sum(de, axis=1, keepdims=True)) / two_h
        pc = pooled - mu
        dc = de - mu
        var = (jnp.sum(pc * pc, axis=1, keepdims=True)
               + jnp.sum(dc * dc, axis=1, keepdims=True)) / two_h
        inv = lax.rsqrt(var + 1e-5)
        pn = pc * inv * ga_ref[...] + ba_ref[...]
        dn = dc * inv * gb_ref[...] + bb_ref[...]
        f = jnp.maximum(
            lax.dot_general(pn, wfa_ref[...], (((1,), (1,)), ((), ())),
                            preferred_element_type=jnp.float32)
            + lax.dot_general(dn, wfb_ref[...], (((1,), (1,)), ((), ())),
                              preferred_element_type=jnp.float32)
            + bf_ref[...], 0.0)
        out_ref[...] = lax.dot_general(
            f, wto_ref[...], (((1,), (1,)), ((), ())),
            preferred_element_type=jnp.float32) + bto_ref[...]


def _tc_poolhead(agg, hs, dinvc, b, batchr, doc, Wd, bd, ga, gb, ba, bb,
                 Wfa, Wfb, bf, Wto, bto):
    DOC = doc.shape[1]
    return pl.pallas_call(
        _tc_poolhead_body,
        grid=(NBLK,),
        in_specs=[
            pl.BlockSpec((NC, BLK, H), lambda i: (0, i, 0)),
            _rows(H),
            _rows(16),
            _full((1, H)),
            pl.BlockSpec((1, 1, BLK), lambda i: (i, 0, 0)),
            _full((G, DOC)), _full((H, DOC)), _full((1, H)),
            _full((1, H)), _full((1, H)), _full((1, H)), _full((1, H)),
            _full((H, H)), _full((H, H)), _full((1, H)),
            _full((16, H)), _full((1, 16)),
        ],
        out_specs=_full((G, 16)),
        out_shape=jax.ShapeDtypeStruct((G, 16), jnp.float32),
        scratch_shapes=[
            pltpu.VMEM((G, H), jnp.float32),
            pltpu.VMEM((G, H), jnp.float32),
        ],
    )(agg, hs, dinvc, b, batchr, doc, Wd, bd, ga, gb, ba, bb,
      Wfa, Wfb, bf, Wto, bto)


def kernel(x, edge_index, batch, doc_features, W1, b1, W2, b2, W3, b3,
           Wp, bp, Wd, bd, gamma, beta, Wf, bf, Wt, bt, Wtm, btm):
    f32 = jnp.float32
    ei4 = edge_index.astype(jnp.int32).reshape(2, NW, NCHUNK, CH)
    ei2 = edge_index.astype(jnp.int32).reshape(2, NW, EPW)
    batchr = batch.astype(jnp.int32).reshape(NBLK, 1, BLK)
    zeros128 = jnp.zeros((RPT, H), f32)

    b1r = b1.reshape(1, H)
    b2r = b2.reshape(1, H)
    b3r = b3.reshape(1, H)
    bpr = bp.reshape(1, H)
    bdr = bd.reshape(1, H)
    bfr = bf.reshape(1, H)
    ga, gb = gamma[:H].reshape(1, H), gamma[H:].reshape(1, H)
    ba, bb = beta[:H].reshape(1, H), beta[H:].reshape(1, H)
    Wfa, Wfb = Wf[:, :H], Wf[:, H:]
    Wto = jnp.concatenate(
        [Wt, Wtm, jnp.zeros((16 - Wt.shape[0] - 1, H), f32)], axis=0)
    bto = jnp.concatenate(
        [bt, btm, jnp.zeros((16 - bt.shape[0] - 1,), f32)]).reshape(1, 16)

    degp = _sc_degree(ei2)
    hs1, xp, dinvc = _tc_pre(x, W1, Wp, bpr, degp)
    agg1 = _sc_aggregate(hs1, ei4, zeros128)
    hs2 = _tc_mid(agg1, hs1, dinvc, b1r, W2, xp, with_xp=True)
    agg2 = _sc_aggregate(hs2, ei4, zeros128)
    hs3 = _tc_mid(agg2, hs2, dinvc, b2r, W3, xp, with_xp=False)
    agg3 = _sc_aggregate(hs3, ei4, zeros128)
    out = _tc_poolhead(agg3, hs3, dinvc, b3r, batchr, doc_features, Wd,
                       bdr, ga, gb, ba, bb, Wfa, Wfb, bfr, Wto, bto)
    task = out[:, :10]
    time = out[:, 10:11]
    return (task, time)


# speedup vs baseline: 1.1950x; 1.0014x over previous
import functools

import jax
import jax.numpy as jnp
from jax import lax
from jax.experimental import pallas as pl
from jax.experimental.pallas import tpu as pltpu
from jax.experimental.pallas import tpu_sc as plsc

N = 10000
E = 320000
D = 128
H = 128
G = 64

NC = 2
NS = 16
NW = NC * NS
EPW = E // NW
CH = 125
NCHUNK = EPW // CH
IW = 8
NWIN = NCHUNK // IW
NP = 10240
RPT = NP // NS

_mesh_cache = []


def _MESH():
    if not _mesh_cache:
        _mesh_cache.append(plsc.VectorSubcoreMesh(
            core_axis_name="c", subcore_axis_name="s",
            num_cores=NC, num_subcores=NS))
    return _mesh_cache[0]


def _sc_degree_body(ei2, out, idx1, cnt_v, seg_v, seg2_v, rep_v):
    cid = lax.axis_index("c")
    sid = lax.axis_index("s")
    wid = cid * NS + sid
    for i in range(NP // 16):
        cnt_v[pl.ds(i * 16, 16)] = jnp.zeros((16,), jnp.float32)
    pltpu.sync_copy(ei2.at[1, wid], idx1)
    ones = jnp.ones((16,), jnp.float32)
    for i in range(EPW // 16):
        iv = idx1[pl.ds(i * 16, 16)]
        plsc.addupdate_scatter(cnt_v, [iv], ones)
    pltpu.sync_copy(cnt_v, seg_v.at[sid])
    plsc.subcore_barrier()
    pltpu.sync_copy(seg_v.at[:, pl.ds(sid * RPT, RPT)], seg2_v)
    col0 = jnp.zeros((16,), jnp.int32)
    for g in range(RPT // 64):
        for j in range(4):
            base = g * 64 + j * 16
            t = seg2_v[0, pl.ds(base, 16)]
            for tt in range(1, NS):
                t = t + seg2_v[tt, pl.ds(base, 16)]
            ridx = lax.iota(jnp.int32, 16) + j * 16
            plsc.store_scatter(rep_v, [ridx, col0], t)
        pltpu.sync_copy(
            rep_v, out.at[cid, pl.ds(sid * RPT + g * 64, 64)])


def _sc_degree(ei2):
    return pl.kernel(
        _sc_degree_body,
        out_type=jax.ShapeDtypeStruct((NC, NP, H), jnp.float32),
        mesh=_MESH(),
        compiler_params=pltpu.CompilerParams(needs_layout_passes=False),
        scratch_types=[
            pltpu.VMEM((EPW,), jnp.int32),
            pltpu.VMEM((NP,), jnp.float32),
            pltpu.VMEM_SHARED((NS, NP), jnp.float32),
            pltpu.VMEM((NS, RPT), jnp.float32),
            pltpu.VMEM((64, H), jnp.float32),
        ],
    )(ei2)


NBUF = 2


def _sc_aggregate_body(hs, ei4, zeros128, out,
                       sidx_v, didx_v, rows0, rows1, acc, gsems, ssems, isem):
    cid = lax.axis_index("c")
    sid = lax.axis_index("s")
    wid = cid * NS + sid
    rows = (rows0, rows1)
    pltpu.sync_copy(zeros128, acc.at[pl.ds(sid * RPT, RPT)])
    pltpu.sync_copy(ei4.at[0, wid, pl.ds(0, IW)], sidx_v.at[0])
    pltpu.sync_copy(ei4.at[1, wid, pl.ds(0, IW)], didx_v.at[0])
    pltpu.async_copy(ei4.at[0, wid, pl.ds(IW, IW)], sidx_v.at[1], isem)
    pltpu.async_copy(ei4.at[1, wid, pl.ds(IW, IW)], didx_v.at[1], isem)
    plsc.subcore_barrier()

    def gath(p, l, b):
        pltpu.async_copy(hs.at[sidx_v.at[p, l]], rows[b], gsems.at[b])

    def gath_wait(b):
        pltpu.make_async_copy(hs.at[sidx_v.at[0, 0]], rows[b],
                              gsems.at[b]).wait()

    def scat(p, l, b):
        pltpu.async_copy(rows[b], acc.at[didx_v.at[p, l]], ssems.at[b],
                         add=True)

    def scat_wait(b):
        pltpu.make_async_copy(rows[b], acc.at[didx_v.at[0, 0]],
                              ssems.at[b]).wait()

    def idx_load(w, p):
        pltpu.async_copy(ei4.at[0, wid, pl.ds(w * IW, IW)], sidx_v.at[p], isem)
        pltpu.async_copy(ei4.at[1, wid, pl.ds(w * IW, IW)], didx_v.at[p], isem)

    def idx_wait(p):
        pltpu.make_async_copy(ei4.at[0, wid, pl.ds(0, IW)], sidx_v.at[p],
                              isem).wait()
        pltpu.make_async_copy(ei4.at[1, wid, pl.ds(0, IW)], didx_v.at[p],
                              isem).wait()

    gath(0, 0, 0)
    for w in range(NWIN):
        p = w % 2
        if w >= 1:
            scat_wait(1)
            if w <= NWIN - 2:
                idx_load(w + 1, 1 - p)
        gath(p, 1, 1)
        gath_wait(0)
        scat(p, 0, 0)

        def pair(ii, carry, p=p):
            l0 = ii * 2 + 1
            scat_wait(0)
            gath(p, l0 + 1, 0)
            gath_wait(1)
            scat(p, l0, 1)
            scat_wait(1)
            gath(p, l0 + 2, 1)
            gath_wait(0)
            scat(p, l0 + 1, 0)
            return carry

        lax.fori_loop(0, (IW - 2) // 2, pair, 0)

        scat_wait(0)
        if w < NWIN - 1:
            idx_wait(1 - p)
            gath(1 - p, 0, 0)
        gath_wait(1)
        scat(p, IW - 1, 1)

    scat_wait(1)
    plsc.subcore_barrier()
    pltpu.sync_copy(acc.at[pl.ds(sid * RPT, RPT)],
                    out.at[cid, pl.ds(sid * RPT, RPT)])


def _sc_aggregate(hs, ei4, zeros128):
    return pl.kernel(
        _sc_aggregate_body,
        out_type=jax.ShapeDtypeStruct((NC, NP, H), jnp.float32),
        mesh=_MESH(),
        scratch_types=[
            pltpu.VMEM((2, IW, CH), jnp.int32),
            pltpu.VMEM((2, IW, CH), jnp.int32),
            pltpu.VMEM((CH, H), jnp.float32),
            pltpu.VMEM((CH, H), jnp.float32),
            pltpu.VMEM_SHARED((NP, H), jnp.float32),
            pltpu.SemaphoreType.DMA((NBUF,)),
            pltpu.SemaphoreType.DMA((NBUF,)),
            pltpu.SemaphoreType.DMA,
        ],
    )(hs, ei4, zeros128)


BLK = 5000
NBLK = N // BLK
_full = lambda shape: pl.BlockSpec(shape, lambda i: (0,) * len(shape))
_rows = lambda w: pl.BlockSpec((BLK, w), lambda i: (i, 0))


def _tc_pre_body(x_ref, w1_ref, wp_ref, bp_ref, degp_ref,
                 hs1_ref, xp_ref, dinvc_ref):
    deg = degp_ref[0, :, 0:1] + degp_ref[1, :, 0:1] + 1.0
    dinv = lax.rsqrt(deg)
    x = x_ref[...]
    h = lax.dot_general(x, w1_ref[...], (((1,), (1,)), ((), ())),
                        preferred_element_type=jnp.float32)
    hs1_ref[...] = dinv * h
    xp_ref[...] = lax.dot_general(x, wp_ref[...], (((1,), (1,)), ((), ())),
                                  preferred_element_type=jnp.float32) + bp_ref[...]
    dinvc_ref[...] = jnp.broadcast_to(dinv, (BLK, 16))


def _tc_pre(x, W1, Wp, bp, degp):
    return pl.pallas_call(
        _tc_pre_body,
        grid=(NBLK,),
        in_specs=[
            _rows(D),
            _full((H, D)),
            _full((H, D)),
            _full((1, H)),
            pl.BlockSpec((NC, BLK, H), lambda i: (0, i, 0)),
        ],
        out_specs=[_rows(H), _rows(H), _rows(16)],
        out_shape=[
            jax.ShapeDtypeStruct((N, H), jnp.float32),
            jax.ShapeDtypeStruct((N, H), jnp.float32),
            jax.ShapeDtypeStruct((N, 16), jnp.float32),
        ],
    )(x, W1, Wp, bp, degp)


def _tc_mid_body(agg_ref, hs_ref, dinvc_ref, b_ref, wn_ref, xp_ref,
                 out_ref, *, with_xp):
    dinv = dinvc_ref[:, 0:1]
    t = agg_ref[0] + agg_ref[1] + hs_ref[...]
    t = jnp.maximum(dinv * t + b_ref[...], 0.0)
    if with_xp:
        t = t + xp_ref[...]
    else:
        t = t + t
    out_ref[...] = dinv * lax.dot_general(
        t, wn_ref[...], (((1,), (1,)), ((), ())),
        preferred_element_type=jnp.float32)


def _tc_mid(agg, hs, dinvc, b, Wn, xp, with_xp):
    return pl.pallas_call(
        functools.partial(_tc_mid_body, with_xp=with_xp),
        grid=(NBLK,),
        in_specs=[
            pl.BlockSpec((NC, BLK, H), lambda i: (0, i, 0)),
            _rows(H),
            _rows(16),
            _full((1, H)),
            _full((H, H)),
            _rows(H),
        ],
        out_specs=_rows(H),
        out_shape=jax.ShapeDtypeStruct((N, H), jnp.float32),
    )(agg, hs, dinvc, b, Wn, xp)


def _tc_poolhead_body(agg_ref, hs_ref, dinvc_ref, b_ref, batch_ref,
                      doc_ref, wd_ref, bd_ref, ga_ref, gb_ref, ba_ref,
                      bb_ref, wfa_ref, wfb_ref, bf_ref, wto_ref, bto_ref,
                      out_ref, sums_ref, cnt_ref):
    dinv = dinvc_ref[:, 0:1]
    t = agg_ref[0] + agg_ref[1] + hs_ref[...]
    h3 = jnp.maximum(dinv * t + b_ref[...], 0.0)
    h3 = h3 + h3
    bblk = batch_ref[0]
    giota = lax.broadcasted_iota(jnp.int32, (G, BLK), 0)
    oh = (bblk == giota).astype(jnp.float32)
    s = lax.dot_general(oh, h3, (((1,), (0,)), ((), ())),
                        preferred_element_type=jnp.float32)
    c = jnp.broadcast_to(jnp.sum(oh, axis=1, keepdims=True), (G, H))

    @pl.when(pl.program_id(0) == 0)
    def _():
        sums_ref[...] = jnp.zeros_like(sums_ref)
        cnt_ref[...] = jnp.zeros_like(cnt_ref)

    sums_ref[...] += s
    cnt_ref[...] += c

    @pl.when(pl.program_id(0) == NBLK - 1)
    def _():
        pooled = sums_ref[...] / jnp.maximum(cnt_ref[...], 1.0)
        de = jnp.maximum(
            lax.dot_general(doc_ref[...], wd_ref[...],
                            (((1,), (1,)), ((), ())),
                            preferred_element_type=jnp.float32)
            + bd_ref[...], 0.0)
        two_h = 2.0 * H
        mu = (jnp.sum(pooled, axis=1, keepdims=True)
              + jnp.sum(de, axis=1, keepdims=True)) / two_h
        pc = pooled - mu
        dc = de - mu
        var = (jnp.sum(pc * pc, axis=1, keepdims=True)
               + jnp.sum(dc * dc, axis=1, keepdims=True)) / two_h
        inv = lax.rsqrt(var + 1e-5)
        pn = pc * inv * ga_ref[...] + ba_ref[...]
        dn = dc * inv * gb_ref[...] + bb_ref[...]
        f = jnp.maximum(
            lax.dot_general(pn, wfa_ref[...], (((1,), (1,)), ((), ())),
                            preferred_element_type=jnp.float32)
            + lax.dot_general(dn, wfb_ref[...], (((1,), (1,)), ((), ())),
                              preferred_element_type=jnp.float32)
            + bf_ref[...], 0.0)
        out_ref[...] = lax.dot_general(
            f, wto_ref[...], (((1,), (1,)), ((), ())),
            preferred_element_type=jnp.float32) + bto_ref[...]


def _tc_poolhead(agg, hs, dinvc, b, batchr, doc, Wd, bd, ga, gb, ba, bb,
                 Wfa, Wfb, bf, Wto, bto):
    DOC = doc.shape[1]
    return pl.pallas_call(
        _tc_poolhead_body,
        grid=(NBLK,),
        in_specs=[
            pl.BlockSpec((NC, BLK, H), lambda i: (0, i, 0)),
            _rows(H),
            _rows(16),
            _full((1, H)),
            pl.BlockSpec((1, 1, BLK), lambda i: (i, 0, 0)),
            _full((G, DOC)), _full((H, DOC)), _full((1, H)),
            _full((1, H)), _full((1, H)), _full((1, H)), _full((1, H)),
            _full((H, H)), _full((H, H)), _full((1, H)),
            _full((16, H)), _full((1, 16)),
        ],
        out_specs=_full((G, 16)),
        out_shape=jax.ShapeDtypeStruct((G, 16), jnp.float32),
        scratch_shapes=[
            pltpu.VMEM((G, H), jnp.float32),
            pltpu.VMEM((G, H), jnp.float32),
        ],
    )(agg, hs, dinvc, b, batchr, doc, Wd, bd, ga, gb, ba, bb,
      Wfa, Wfb, bf, Wto, bto)


def kernel(x, edge_index, batch, doc_features, W1, b1, W2, b2, W3, b3,
           Wp, bp, Wd, bd, gamma, beta, Wf, bf, Wt, bt, Wtm, btm):
    f32 = jnp.float32
    ei4 = edge_index.astype(jnp.int32).reshape(2, NW, NCHUNK, CH)
    ei2 = edge_index.astype(jnp.int32).reshape(2, NW, EPW)
    batchr = batch.astype(jnp.int32).reshape(NBLK, 1, BLK)
    zeros128 = jnp.zeros((RPT, H), f32)

    b1r = b1.reshape(1, H)
    b2r = b2.reshape(1, H)
    b3r = b3.reshape(1, H)
    bpr = bp.reshape(1, H)
    bdr = bd.reshape(1, H)
    bfr = bf.reshape(1, H)
    ga, gb = gamma[:H].reshape(1, H), gamma[H:].reshape(1, H)
    ba, bb = beta[:H].reshape(1, H), beta[H:].reshape(1, H)
    Wfa, Wfb = Wf[:, :H], Wf[:, H:]
    Wto = jnp.concatenate(
        [Wt, Wtm, jnp.zeros((16 - Wt.shape[0] - 1, H), f32)], axis=0)
    bto = jnp.concatenate(
        [bt, btm, jnp.zeros((16 - bt.shape[0] - 1,), f32)]).reshape(1, 16)

    degp = _sc_degree(ei2)
    hs1, xp, dinvc = _tc_pre(x, W1, Wp, bpr, degp)
    agg1 = _sc_aggregate(hs1, ei4, zeros128)
    hs2 = _tc_mid(agg1, hs1, dinvc, b1r, W2, xp, with_xp=True)
    agg2 = _sc_aggregate(hs2, ei4, zeros128)
    hs3 = _tc_mid(agg2, hs2, dinvc, b2r, W3, xp, with_xp=False)
    agg3 = _sc_aggregate(hs3, ei4, zeros128)
    out = _tc_poolhead(agg3, hs3, dinvc, b3r, batchr, doc_features, Wd,
                       bdr, ga, gb, ba, bb, Wfa, Wfb, bfr, Wto, bto)
    task = out[:, :10]
    time = out[:, 10:11]
    return (task, time)


# speedup vs baseline: 1.2195x; 1.0206x over previous
import functools

import jax
import jax.numpy as jnp
from jax import lax
from jax.experimental import pallas as pl
from jax.experimental.pallas import tpu as pltpu
from jax.experimental.pallas import tpu_sc as plsc

N = 10000
E = 320000
D = 128
H = 128
G = 64

NC = 2
NS = 16
NW = NC * NS
EPW = E // NW
CH = 50
NCHUNK = EPW // CH
IW = 8
NWIN = NCHUNK // IW
NP = 10240
RPT = NP // NS

_mesh_cache = []


def _MESH():
    if not _mesh_cache:
        _mesh_cache.append(plsc.VectorSubcoreMesh(
            core_axis_name="c", subcore_axis_name="s",
            num_cores=NC, num_subcores=NS))
    return _mesh_cache[0]


def _sc_degree_body(ei2, out, idx1, cnt_v, seg_v, seg2_v, rep_v):
    cid = lax.axis_index("c")
    sid = lax.axis_index("s")
    wid = cid * NS + sid
    for i in range(NP // 16):
        cnt_v[pl.ds(i * 16, 16)] = jnp.zeros((16,), jnp.float32)
    pltpu.sync_copy(ei2.at[1, wid], idx1)
    ones = jnp.ones((16,), jnp.float32)
    for i in range(EPW // 16):
        iv = idx1[pl.ds(i * 16, 16)]
        plsc.addupdate_scatter(cnt_v, [iv], ones)
    pltpu.sync_copy(cnt_v, seg_v.at[sid])
    plsc.subcore_barrier()
    pltpu.sync_copy(seg_v.at[:, pl.ds(sid * RPT, RPT)], seg2_v)
    col0 = jnp.zeros((16,), jnp.int32)
    for g in range(RPT // 64):
        for j in range(4):
            base = g * 64 + j * 16
            t = seg2_v[0, pl.ds(base, 16)]
            for tt in range(1, NS):
                t = t + seg2_v[tt, pl.ds(base, 16)]
            ridx = lax.iota(jnp.int32, 16) + j * 16
            plsc.store_scatter(rep_v, [ridx, col0], t)
        pltpu.sync_copy(
            rep_v, out.at[cid, pl.ds(sid * RPT + g * 64, 64)])


def _sc_degree(ei2):
    return pl.kernel(
        _sc_degree_body,
        out_type=jax.ShapeDtypeStruct((NC, NP, H), jnp.float32),
        mesh=_MESH(),
        compiler_params=pltpu.CompilerParams(needs_layout_passes=False),
        scratch_types=[
            pltpu.VMEM((EPW,), jnp.int32),
            pltpu.VMEM((NP,), jnp.float32),
            pltpu.VMEM_SHARED((NS, NP), jnp.float32),
            pltpu.VMEM((NS, RPT), jnp.float32),
            pltpu.VMEM((64, H), jnp.float32),
        ],
    )(ei2)


NBUF = 4


def _sc_aggregate_body(hs, ei4, zeros128, out,
                       sidx_v, didx_v, rows0, rows1, rows2, rows3,
                       acc, gsems, ssems, isem):
    cid = lax.axis_index("c")
    sid = lax.axis_index("s")
    wid = cid * NS + sid
    rows = (rows0, rows1, rows2, rows3)
    pltpu.sync_copy(zeros128, acc.at[pl.ds(sid * RPT, RPT)])
    pltpu.sync_copy(ei4.at[0, wid, pl.ds(0, IW)], sidx_v.at[0])
    pltpu.sync_copy(ei4.at[1, wid, pl.ds(0, IW)], didx_v.at[0])
    pltpu.async_copy(ei4.at[0, wid, pl.ds(IW, IW)], sidx_v.at[1], isem)
    pltpu.async_copy(ei4.at[1, wid, pl.ds(IW, IW)], didx_v.at[1], isem)
    plsc.subcore_barrier()

    def gath(p, l, b):
        pltpu.async_copy(hs.at[sidx_v.at[p, l]], rows[b], gsems.at[b])

    def gath_wait(b):
        pltpu.make_async_copy(hs.at[sidx_v.at[0, 0]], rows[b],
                              gsems.at[b]).wait()

    def scat(p, l, b):
        pltpu.async_copy(rows[b], acc.at[didx_v.at[p, l]], ssems.at[b],
                         add=True)

    def scat_wait(b):
        pltpu.make_async_copy(rows[b], acc.at[didx_v.at[0, 0]],
                              ssems.at[b]).wait()

    def idx_load(wnext, p):
        off = pl.multiple_of(wnext * IW, IW)
        pltpu.async_copy(ei4.at[0, wid, pl.ds(off, IW)], sidx_v.at[p], isem)
        pltpu.async_copy(ei4.at[1, wid, pl.ds(off, IW)], didx_v.at[p], isem)

    def idx_wait(p):
        pltpu.make_async_copy(ei4.at[0, wid, pl.ds(0, IW)], sidx_v.at[p],
                              isem).wait()
        pltpu.make_async_copy(ei4.at[1, wid, pl.ds(0, IW)], didx_v.at[p],
                              isem).wait()

    def slot(w_dyn, p, l, first=False, last=False, no_idx_load=False):
        b = l % 4
        b2 = (l + 2) % 4
        if not first or l >= 2:
            scat_wait(b2)
        if l == 1 and not last and not no_idx_load:
            idx_load(w_dyn + 1, 1 - p)
        if l == 6 and not last:
            idx_wait(1 - p)
        if not (last and l >= 6):
            if l >= 6:
                gath(1 - p, (l + 2) % 8, b2)
            else:
                gath(p, l + 2, b2)
        gath_wait(b)
        scat(p, l, b)

    gath(0, 0, 0)
    gath(0, 1, 1)
    for l in range(IW):
        slot(0, 0, l, first=True, no_idx_load=True)

    def pair(jj, carry):
        w0 = jj * 2 + 1
        for l in range(IW):
            slot(w0, 1, l)
        for l in range(IW):
            slot(w0 + 1, 0, l)
        return carry

    lax.fori_loop(0, (NWIN - 3) // 2, pair, 0)

    for l in range(IW):
        slot(NWIN - 2, 1, l)
    for l in range(IW):
        slot(NWIN - 1, 0, l, last=True)
    scat_wait((NCHUNK - 2) % 4)
    scat_wait((NCHUNK - 1) % 4)
    plsc.subcore_barrier()
    pltpu.sync_copy(acc.at[pl.ds(sid * RPT, RPT)],
                    out.at[cid, pl.ds(sid * RPT, RPT)])


def _sc_aggregate(hs, ei4, zeros128):
    return pl.kernel(
        _sc_aggregate_body,
        out_type=jax.ShapeDtypeStruct((NC, NP, H), jnp.float32),
        mesh=_MESH(),
        scratch_types=[
            pltpu.VMEM((2, IW, CH), jnp.int32),
            pltpu.VMEM((2, IW, CH), jnp.int32),
            pltpu.VMEM((CH, H), jnp.float32),
            pltpu.VMEM((CH, H), jnp.float32),
            pltpu.VMEM((CH, H), jnp.float32),
            pltpu.VMEM((CH, H), jnp.float32),
            pltpu.VMEM_SHARED((NP, H), jnp.float32),
            pltpu.SemaphoreType.DMA((NBUF,)),
            pltpu.SemaphoreType.DMA((NBUF,)),
            pltpu.SemaphoreType.DMA,
        ],
    )(hs, ei4, zeros128)


BLK = 5000
NBLK = N // BLK
_full = lambda shape: pl.BlockSpec(shape, lambda i: (0,) * len(shape))
_rows = lambda w: pl.BlockSpec((BLK, w), lambda i: (i, 0))


def _tc_pre_body(x_ref, w1_ref, wp_ref, bp_ref, degp_ref,
                 hs1_ref, xp_ref, dinvc_ref):
    deg = degp_ref[0, :, 0:1] + degp_ref[1, :, 0:1] + 1.0
    dinv = lax.rsqrt(deg)
    x = x_ref[...]
    h = lax.dot_general(x, w1_ref[...], (((1,), (1,)), ((), ())),
                        preferred_element_type=jnp.float32)
    hs1_ref[...] = dinv * h
    xp_ref[...] = lax.dot_general(x, wp_ref[...], (((1,), (1,)), ((), ())),
                                  preferred_element_type=jnp.float32) + bp_ref[...]
    dinvc_ref[...] = jnp.broadcast_to(dinv, (BLK, 16))


def _tc_pre(x, W1, Wp, bp, degp):
    return pl.pallas_call(
        _tc_pre_body,
        grid=(NBLK,),
        in_specs=[
            _rows(D),
            _full((H, D)),
            _full((H, D)),
            _full((1, H)),
            pl.BlockSpec((NC, BLK, H), lambda i: (0, i, 0)),
        ],
        out_specs=[_rows(H), _rows(H), _rows(16)],
        out_shape=[
            jax.ShapeDtypeStruct((N, H), jnp.float32),
            jax.ShapeDtypeStruct((N, H), jnp.float32),
            jax.ShapeDtypeStruct((N, 16), jnp.float32),
        ],
    )(x, W1, Wp, bp, degp)


def _tc_mid_body(agg_ref, hs_ref, dinvc_ref, b_ref, wn_ref, xp_ref,
                 out_ref, *, with_xp):
    dinv = dinvc_ref[:, 0:1]
    t = agg_ref[0] + agg_ref[1] + hs_ref[...]
    t = jnp.maximum(dinv * t + b_ref[...], 0.0)
    if with_xp:
        t = t + xp_ref[...]
    else:
        t = t + t
    out_ref[...] = dinv * lax.dot_general(
        t, wn_ref[...], (((1,), (1,)), ((), ())),
        preferred_element_type=jnp.float32)


def _tc_mid(agg, hs, dinvc, b, Wn, xp, with_xp):
    return pl.pallas_call(
        functools.partial(_tc_mid_body, with_xp=with_xp),
        grid=(NBLK,),
        in_specs=[
            pl.BlockSpec((NC, BLK, H), lambda i: (0, i, 0)),
            _rows(H),
            _rows(16),
            _full((1, H)),
            _full((H, H)),
            _rows(H),
        ],
        out_specs=_rows(H),
        out_shape=jax.ShapeDtypeStruct((N, H), jnp.float32),
    )(agg, hs, dinvc, b, Wn, xp)


def _tc_poolhead_body(agg_ref, hs_ref, dinvc_ref, b_ref, batch_ref,
                      doc_ref, wd_ref, bd_ref, ga_ref, gb_ref, ba_ref,
                      bb_ref, wfa_ref, wfb_ref, bf_ref, wto_ref, bto_ref,
                      out_ref, sums_ref, cnt_ref):
    dinv = dinvc_ref[:, 0:1]
    t = agg_ref[0] + agg_ref[1] + hs_ref[...]
    h3 = jnp.maximum(dinv * t + b_ref[...], 0.0)
    h3 = h3 + h3
    bblk = batch_ref[0]
    giota = lax.broadcasted_iota(jnp.int32, (G, BLK), 0)
    oh = (bblk == giota).astype(jnp.float32)
    s = lax.dot_general(oh, h3, (((1,), (0,)), ((), ())),
                        preferred_element_type=jnp.float32)
    c = jnp.broadcast_to(jnp.sum(oh, axis=1, keepdims=True), (G, H))

    @pl.when(pl.program_id(0) == 0)
    def _():
        sums_ref[...] = jnp.zeros_like(sums_ref)
        cnt_ref[...] = jnp.zeros_like(cnt_ref)

    sums_ref[...] += s
    cnt_ref[...] += c

    @pl.when(pl.program_id(0) == NBLK - 1)
    def _():
        pooled = sums_ref[...] / jnp.maximum(cnt_ref[...], 1.0)
        de = jnp.maximum(
            lax.dot_general(doc_ref[...], wd_ref[...],
                            (((1,), (1,)), ((), ())),
                            preferred_element_type=jnp.float32)
            + bd_ref[...], 0.0)
        two_h = 2.0 * H
        mu = (jnp.sum(pooled, axis=1, keepdims=True)
              + jnp.sum(de, axis=1, keepdims=True)) / two_h
        pc = pooled - mu
        dc = de - mu
        var = (jnp.sum(pc * pc, axis=1, keepdims=True)
               + jnp.sum(dc * dc, axis=1, keepdims=True)) / two_h
        inv = lax.rsqrt(var + 1e-5)
        pn = pc * inv * ga_ref[...] + ba_ref[...]
        dn = dc * inv * gb_ref[...] + bb_ref[...]
        f = jnp.maximum(
            lax.dot_general(pn, wfa_ref[...], (((1,), (1,)), ((), ())),
                            preferred_element_type=jnp.float32)
            + lax.dot_general(dn, wfb_ref[...], (((1,), (1,)), ((), ())),
                              preferred_element_type=jnp.float32)
            + bf_ref[...], 0.0)
        out_ref[...] = lax.dot_general(
            f, wto_ref[...], (((1,), (1,)), ((), ())),
            preferred_element_type=jnp.float32) + bto_ref[...]


def _tc_poolhead(agg, hs, dinvc, b, batchr, doc, Wd, bd, ga, gb, ba, bb,
                 Wfa, Wfb, bf, Wto, bto):
    DOC = doc.shape[1]
    return pl.pallas_call(
        _tc_poolhead_body,
        grid=(NBLK,),
        in_specs=[
            pl.BlockSpec((NC, BLK, H), lambda i: (0, i, 0)),
            _rows(H),
            _rows(16),
            _full((1, H)),
            pl.BlockSpec((1, 1, BLK), lambda i: (i, 0, 0)),
            _full((G, DOC)), _full((H, DOC)), _full((1, H)),
            _full((1, H)), _full((1, H)), _full((1, H)), _full((1, H)),
            _full((H, H)), _full((H, H)), _full((1, H)),
            _full((16, H)), _full((1, 16)),
        ],
        out_specs=_full((G, 16)),
        out_shape=jax.ShapeDtypeStruct((G, 16), jnp.float32),
        scratch_shapes=[
            pltpu.VMEM((G, H), jnp.float32),
            pltpu.VMEM((G, H), jnp.float32),
        ],
    )(agg, hs, dinvc, b, batchr, doc, Wd, bd, ga, gb, ba, bb,
      Wfa, Wfb, bf, Wto, bto)


def kernel(x, edge_index, batch, doc_features, W1, b1, W2, b2, W3, b3,
           Wp, bp, Wd, bd, gamma, beta, Wf, bf, Wt, bt, Wtm, btm):
    f32 = jnp.float32
    ei4 = edge_index.astype(jnp.int32).reshape(2, NW, NCHUNK, CH)
    ei2 = edge_index.astype(jnp.int32).reshape(2, NW, EPW)
    batchr = batch.astype(jnp.int32).reshape(NBLK, 1, BLK)
    zeros128 = jnp.zeros((RPT, H), f32)

    b1r = b1.reshape(1, H)
    b2r = b2.reshape(1, H)
    b3r = b3.reshape(1, H)
    bpr = bp.reshape(1, H)
    bdr = bd.reshape(1, H)
    bfr = bf.reshape(1, H)
    ga, gb = gamma[:H].reshape(1, H), gamma[H:].reshape(1, H)
    ba, bb = beta[:H].reshape(1, H), beta[H:].reshape(1, H)
    Wfa, Wfb = Wf[:, :H], Wf[:, H:]
    Wto = jnp.concatenate(
        [Wt, Wtm, jnp.zeros((16 - Wt.shape[0] - 1, H), f32)], axis=0)
    bto = jnp.concatenate(
        [bt, btm, jnp.zeros((16 - bt.shape[0] - 1,), f32)]).reshape(1, 16)

    degp = _sc_degree(ei2)
    hs1, xp, dinvc = _tc_pre(x, W1, Wp, bpr, degp)
    agg1 = _sc_aggregate(hs1, ei4, zeros128)
    hs2 = _tc_mid(agg1, hs1, dinvc, b1r, W2, xp, with_xp=True)
    agg2 = _sc_aggregate(hs2, ei4, zeros128)
    hs3 = _tc_mid(agg2, hs2, dinvc, b2r, W3, xp, with_xp=False)
    agg3 = _sc_aggregate(hs3, ei4, zeros128)
    out = _tc_poolhead(agg3, hs3, dinvc, b3r, batchr, doc_features, Wd,
                       bdr, ga, gb, ba, bb, Wfa, Wfb, bfr, Wto, bto)
    task = out[:, :10]
    time = out[:, 10:11]
    return (task, time)
